# baseline (speedup 1.0000x reference)
import jax
import jax.numpy as jnp
from jax.experimental import pallas as pl
from jax.experimental.pallas import tpu as pltpu

N = 128
H = 64
C = 8
NCH = N // C


def _mpn_body(fa_ref, fb_ref, adj_ref, adjT_ref, WaT_ref, WbT_ref, W0T_ref,
              W1T_ref, WoT_ref, bo_ref, ah_ref, mb_ref,
              ib_ref, ibT_ref, mT_ref, ia_ref, ma_ref, st_ref, acc_ref, xmx_ref,
              fbbuf_ref, fbsem):
    f32 = jnp.float32
    g = pl.program_id(0)

    def fb_copy(slot, ci):
        return pltpu.make_async_copy(
            fb_ref.at[g, pl.ds(ci * C, C)], fbbuf_ref.at[slot], fbsem.at[slot])

    ia = jax.nn.relu(jnp.dot(fa_ref[0], WaT_ref[...], preferred_element_type=f32))
    ia_ref[...] = ia

    fb_copy(0, 0).start()

    def p1(ci, carry):
        A0, X0 = carry
        slot = jax.lax.rem(ci, 2)

        @pl.when(ci + 1 < NCH)
        def _():
            fb_copy(1 - slot, ci + 1).start()

        fb_copy(slot, ci).wait()
        r = ci * C
        fbc = fbbuf_ref[slot]
        msk = adj_ref[0, pl.ds(r, C)]
        masked = fbc * msk[:, :, None]
        pre = jnp.dot(masked.reshape(C * N, -1), WbT_ref[...],
                      preferred_element_type=f32)
        ibc = jax.nn.relu(pre).reshape(C, N, H)
        ib_ref[pl.ds(r, C)] = ibc
        ibT_ref[:, pl.ds(r, C), :] = jnp.transpose(ibc, (1, 0, 2))
        return A0 + jnp.sum(ibc, axis=0), jnp.maximum(X0, jnp.max(ibc, axis=0))

    A0 = jnp.zeros((N, H), f32)
    X0 = jnp.full((N, H), -jnp.inf, f32)
    A0, X0 = jax.lax.fori_loop(0, NCH, p1, (A0, X0))

    ma = ia + A0 * jax.nn.sigmoid(X0)
    ma_ref[...] = ma

    Gm = jax.lax.dot_general(ma, ma, (((1,), (1,)), ((), ())),
                             preferred_element_type=f32)
    RT = Gm * adjT_ref[0]
    STm = jnp.max(RT, axis=1, keepdims=True)
    STe = jnp.exp(RT - STm)
    st_ref[...] = STe / jnp.sum(STe, axis=1, keepdims=True)

    def it1(ci, _):
        r = ci * C
        adjc = adj_ref[0, pl.ds(r, C)]
        ibc = ib_ref[pl.ds(r, C)]
        DT = adjc[:, :, None] * ma_ref[...][None, :, :] - ibc
        mm = jnp.dot(DT.reshape(C * N, H), W0T_ref[...],
                     preferred_element_type=f32).reshape(C, N, H)
        out = jax.nn.relu(ibT_ref[pl.ds(r, C)] + mm) * st_ref[pl.ds(r, C)][:, :, None]
        mT_ref[pl.ds(r, C)] = out
        acc_ref[pl.ds(r, C)] = jnp.sum(out, axis=1)
        xmx_ref[pl.ds(r, C)] = jnp.max(out, axis=1)
        return 0

    jax.lax.fori_loop(0, NCH, it1, 0)

    ma2 = ma_ref[...] + acc_ref[...] * jax.nn.sigmoid(xmx_ref[...])
    ma_ref[...] = ma2

    G2 = jax.lax.dot_general(ma2, ma2, (((1,), (1,)), ((), ())),
                             preferred_element_type=f32)
    R2 = G2 * adj_ref[0]
    S2m = jnp.max(R2, axis=0, keepdims=True)
    S2e = jnp.exp(R2 - S2m)
    st_ref[...] = S2e / jnp.sum(S2e, axis=0, keepdims=True)

    def it2(ci, carry):
        A2, X2 = carry
        r = ci * C
        adjTc = adjT_ref[0, pl.ds(r, C)]
        ma2c = ma_ref[pl.ds(r, C)]
        D2 = adjTc[:, :, None] * ma2c[:, None, :] - mT_ref[pl.ds(r, C)]
        mm = jnp.dot(D2.reshape(C * N, H), W1T_ref[...],
                     preferred_element_type=f32).reshape(C, N, H)
        out = jax.nn.relu(ib_ref[pl.ds(r, C)] + mm) * st_ref[pl.ds(r, C)][:, :, None]
        mb_ref[0, pl.ds(r, C)] = out
        return A2 + jnp.sum(out, axis=0), jnp.maximum(X2, jnp.max(out, axis=0))

    A2 = jnp.zeros((N, H), f32)
    X2 = jnp.full((N, H), -jnp.inf, f32)
    A2, X2 = jax.lax.fori_loop(0, NCH, it2, (A2, X2))

    agg2 = A2 * jax.nn.sigmoid(X2)

    WoT = WoT_ref[...]
    pre_o = (jnp.dot(agg2, WoT[0:H], preferred_element_type=f32)
             + jnp.dot(ma_ref[...], WoT[H:2 * H], preferred_element_type=f32)
             + jnp.dot(ia_ref[...], WoT[2 * H:3 * H], preferred_element_type=f32)
             + bo_ref[...])
    ah_ref[0] = jax.nn.relu(pre_o)


@jax.jit
def kernel(f_atoms, f_bonds, adj, W_i_atom, W_i_bond, W_h_0, W_h_1, W_o, b_o):
    B, S, n, AF = f_atoms.shape
    BF = f_bonds.shape[-1]
    G = B * S
    fa = f_atoms.reshape(G, n, AF)
    fb = f_bonds.reshape(G, n, n, BF)
    a = adj.reshape(G, n, n)
    aT = jnp.swapaxes(a, 1, 2)
    ah, mb = pl.pallas_call(
        _mpn_body,
        grid=(G,),
        in_specs=[
            pl.BlockSpec((1, N, AF), lambda g: (g, 0, 0)),
            pl.BlockSpec(memory_space=pltpu.MemorySpace.HBM),
            pl.BlockSpec((1, N, N), lambda g: (g, 0, 0)),
            pl.BlockSpec((1, N, N), lambda g: (g, 0, 0)),
            pl.BlockSpec((AF, H), lambda g: (0, 0)),
            pl.BlockSpec((BF, H), lambda g: (0, 0)),
            pl.BlockSpec((H, H), lambda g: (0, 0)),
            pl.BlockSpec((H, H), lambda g: (0, 0)),
            pl.BlockSpec((3 * H, H), lambda g: (0, 0)),
            pl.BlockSpec((1, H), lambda g: (0, 0)),
        ],
        out_specs=[
            pl.BlockSpec((1, N, H), lambda g: (g, 0, 0)),
            pl.BlockSpec((1, N, N, H), lambda g: (g, 0, 0, 0)),
        ],
        out_shape=[
            jax.ShapeDtypeStruct((G, N, H), jnp.float32),
            jax.ShapeDtypeStruct((G, N, N, H), jnp.float32),
        ],
        scratch_shapes=[
            pltpu.VMEM((N, N, H), jnp.float32),
            pltpu.VMEM((N, N, H), jnp.float32),
            pltpu.VMEM((N, N, H), jnp.float32),
            pltpu.VMEM((N, H), jnp.float32),
            pltpu.VMEM((N, H), jnp.float32),
            pltpu.VMEM((N, N), jnp.float32),
            pltpu.VMEM((N, H), jnp.float32),
            pltpu.VMEM((N, H), jnp.float32),
            pltpu.VMEM((2, C, N, BF), jnp.float32),
            pltpu.SemaphoreType.DMA((2,)),
        ],
    )(fa, fb, a, aT, W_i_atom.T, W_i_bond.T, W_h_0.T, W_h_1.T, W_o.T,
      b_o.reshape(1, H))
    return ah.reshape(B, S, n, H), mb.reshape(B, S, n, n, H)


# speedup vs baseline: 1.4817x; 1.4817x over previous
import jax
import jax.numpy as jnp
from jax.experimental import pallas as pl
from jax.experimental.pallas import tpu as pltpu

N = 128
H = 64
C = 8
NCH = N // C

_MM_TT = (((0,), (1,)), ((), ()))
_MM_NT = (((1,), (1,)), ((), ()))
_MM_GRAM = (((1,), (1,)), ((), ()))


def _mpn_body(fa_ref, fb_ref, adj_ref, Wa_ref, Wb_ref, W0_ref,
              W1_ref, Wo_ref, bo_ref, ah_ref, mb_ref,
              ib_ref, ibT_ref, mT_ref, adjT_ref, ia_ref, ma_ref, st_ref,
              acc_ref, xmx_ref, fbbuf_ref, fbsem):
    f32 = jnp.float32
    g = pl.program_id(0)

    def fb_copy(slot, ci):
        return pltpu.make_async_copy(
            fb_ref.at[g, :, pl.ds(ci * C, C), :], fbbuf_ref.at[slot],
            fbsem.at[slot])

    fb_copy(0, 0).start()

    adj = adj_ref[0]
    adjT_ref[...] = adj.T

    ia = jax.nn.relu(jax.lax.dot_general(fa_ref[0], Wa_ref[...], _MM_NT,
                                         preferred_element_type=f32))
    ia_ref[...] = ia

    def p1(ci, carry):
        A0, X0 = carry
        slot = jax.lax.rem(ci, 2)

        @pl.when(ci + 1 < NCH)
        def _():
            fb_copy(1 - slot, ci + 1).start()

        fb_copy(slot, ci).wait()
        r = ci * C
        masked = fbbuf_ref[slot] * adj_ref[0, pl.ds(r, C)][None, :, :]
        ibc = jax.nn.relu(jax.lax.dot_general(masked, Wb_ref[...], _MM_TT,
                                              preferred_element_type=f32))
        ib_ref[pl.ds(r, C)] = ibc
        ibT_ref[:, pl.ds(r, C), :] = jnp.transpose(ibc, (1, 0, 2))
        return A0 + jnp.sum(ibc, axis=0), jnp.maximum(X0, jnp.max(ibc, axis=0))

    A0 = jnp.zeros((N, H), f32)
    X0 = jnp.full((N, H), -jnp.inf, f32)
    A0, X0 = jax.lax.fori_loop(0, NCH, p1, (A0, X0))

    ma = ia + A0 * jax.nn.sigmoid(X0)
    ma_ref[...] = ma

    Gm = jax.lax.dot_general(ma, ma, _MM_GRAM, preferred_element_type=f32)
    RT = Gm * adjT_ref[...]
    STm = jnp.max(RT, axis=1, keepdims=True)
    STe = jnp.exp(RT - STm)
    st_ref[...] = STe / jnp.sum(STe, axis=1, keepdims=True)

    def it1(ci, _):
        r = ci * C
        adjc = adj_ref[0, pl.ds(r, C)]
        ibc = ib_ref[pl.ds(r, C)]
        DT = adjc[:, :, None] * ma_ref[...][None, :, :] - ibc
        mm = jax.lax.dot_general(DT.reshape(C * N, H), W0_ref[...], _MM_NT,
                                 preferred_element_type=f32).reshape(C, N, H)
        out = jax.nn.relu(ibT_ref[pl.ds(r, C)] + mm) * st_ref[pl.ds(r, C)][:, :, None]
        mT_ref[pl.ds(r, C)] = out
        acc_ref[pl.ds(r, C)] = jnp.sum(out, axis=1)
        xmx_ref[pl.ds(r, C)] = jnp.max(out, axis=1)
        return 0

    jax.lax.fori_loop(0, NCH, it1, 0)

    ma2 = ma_ref[...] + acc_ref[...] * jax.nn.sigmoid(xmx_ref[...])
    ma_ref[...] = ma2

    G2 = jax.lax.dot_general(ma2, ma2, _MM_GRAM, preferred_element_type=f32)
    R2 = G2 * adj_ref[0]
    S2m = jnp.max(R2, axis=0, keepdims=True)
    S2e = jnp.exp(R2 - S2m)
    st_ref[...] = S2e / jnp.sum(S2e, axis=0, keepdims=True)

    def it2(ci, carry):
        A2, X2 = carry
        r = ci * C
        adjTc = adjT_ref[pl.ds(r, C)]
        ma2c = ma_ref[pl.ds(r, C)]
        D2 = adjTc[:, :, None] * ma2c[:, None, :] - mT_ref[pl.ds(r, C)]
        mm = jax.lax.dot_general(D2.reshape(C * N, H), W1_ref[...], _MM_NT,
                                 preferred_element_type=f32).reshape(C, N, H)
        out = jax.nn.relu(ib_ref[pl.ds(r, C)] + mm) * st_ref[pl.ds(r, C)][:, :, None]
        mb_ref[0, pl.ds(r, C)] = jnp.swapaxes(out, 1, 2)
        return A2 + jnp.sum(out, axis=0), jnp.maximum(X2, jnp.max(out, axis=0))

    A2 = jnp.zeros((N, H), f32)
    X2 = jnp.full((N, H), -jnp.inf, f32)
    A2, X2 = jax.lax.fori_loop(0, NCH, it2, (A2, X2))

    agg2 = A2 * jax.nn.sigmoid(X2)

    Wo = Wo_ref[...]
    pre_o = (jax.lax.dot_general(agg2, Wo[:, 0:H], _MM_NT, preferred_element_type=f32)
             + jax.lax.dot_general(ma_ref[...], Wo[:, H:2 * H], _MM_NT,
                                   preferred_element_type=f32)
             + jax.lax.dot_general(ia_ref[...], Wo[:, 2 * H:3 * H], _MM_NT,
                                   preferred_element_type=f32)
             + bo_ref[...])
    ah_ref[0] = jax.nn.relu(pre_o).T


@jax.jit
def kernel(f_atoms, f_bonds, adj, W_i_atom, W_i_bond, W_h_0, W_h_1, W_o, b_o):
    B, S, n, AF = f_atoms.shape
    BF = f_bonds.shape[-1]
    G = B * S
    fa = f_atoms.reshape(G, n, AF)
    fbT = jnp.transpose(f_bonds, (0, 1, 4, 2, 3)).reshape(G, BF, n, n)
    a = adj.reshape(G, n, n)
    ah, mb = pl.pallas_call(
        _mpn_body,
        grid=(G,),
        in_specs=[
            pl.BlockSpec((1, N, AF), lambda g: (g, 0, 0)),
            pl.BlockSpec(memory_space=pltpu.MemorySpace.HBM),
            pl.BlockSpec((1, N, N), lambda g: (g, 0, 0)),
            pl.BlockSpec((H, AF), lambda g: (0, 0)),
            pl.BlockSpec((H, BF), lambda g: (0, 0)),
            pl.BlockSpec((H, H), lambda g: (0, 0)),
            pl.BlockSpec((H, H), lambda g: (0, 0)),
            pl.BlockSpec((H, 3 * H), lambda g: (0, 0)),
            pl.BlockSpec((1, H), lambda g: (0, 0)),
        ],
        out_specs=[
            pl.BlockSpec((1, H, N), lambda g: (g, 0, 0)),
            pl.BlockSpec((1, N, H, N), lambda g: (g, 0, 0, 0)),
        ],
        out_shape=[
            jax.ShapeDtypeStruct((G, H, N), jnp.float32),
            jax.ShapeDtypeStruct((G, N, H, N), jnp.float32),
        ],
        scratch_shapes=[
            pltpu.VMEM((N, N, H), jnp.float32),
            pltpu.VMEM((N, N, H), jnp.float32),
            pltpu.VMEM((N, N, H), jnp.float32),
            pltpu.VMEM((N, N), jnp.float32),
            pltpu.VMEM((N, H), jnp.float32),
            pltpu.VMEM((N, H), jnp.float32),
            pltpu.VMEM((N, N), jnp.float32),
            pltpu.VMEM((N, H), jnp.float32),
            pltpu.VMEM((N, H), jnp.float32),
            pltpu.VMEM((2, BF, C, N), jnp.float32),
            pltpu.SemaphoreType.DMA((2,)),
        ],
    )(fa, fbT, a, W_i_atom, W_i_bond, W_h_0, W_h_1, W_o, b_o.reshape(1, H))
    ah_l = jnp.transpose(ah, (0, 2, 1)).reshape(B, S, n, H)
    mb_l = jnp.transpose(mb, (0, 1, 3, 2)).reshape(B, S, n, n, H)
    return ah_l, mb_l


# speedup vs baseline: 1.8112x; 1.2224x over previous
import jax
import jax.numpy as jnp
from jax.experimental import pallas as pl
from jax.experimental.pallas import tpu as pltpu

N = 128
H = 64
C = 8
NCH = N // C

_MM_TT = (((0,), (1,)), ((), ()))
_MM_NT = (((1,), (1,)), ((), ()))
_MM_GRAM = (((1,), (1,)), ((), ()))


def _mpn_body(fa_ref, fb_ref, adj_ref, Wa_ref, Wb_ref, W0_ref,
              W1_ref, Wo_ref, bo_ref, ah_ref, mb_ref,
              ib_ref, ibT_ref, mT_ref, adjT_ref, ia_ref, ma_ref, st_ref,
              acc_ref, xmx_ref, fbbuf_ref, fbsem):
    f32 = jnp.float32
    g = pl.program_id(0)
    ng = pl.num_programs(0)
    slot = jax.lax.rem(g, 2)

    def fb_copy(gi, sl):
        return pltpu.make_async_copy(
            fb_ref.at[gi], fbbuf_ref.at[sl], fbsem.at[sl])

    @pl.when(g == 0)
    def _():
        fb_copy(0, 0).start()
        fb_copy(1, 1).start()

    adj = adj_ref[0]
    adjT_ref[...] = adj.T

    ia = jax.nn.relu(jax.lax.dot_general(fa_ref[0], Wa_ref[...], _MM_NT,
                                         preferred_element_type=f32))
    ia_ref[...] = ia

    fb_copy(g, slot).wait()

    def p1(ci, carry):
        A0, X0 = carry
        r = ci * C
        masked = fbbuf_ref[slot, :, pl.ds(r, C), :] * adj_ref[0, pl.ds(r, C)][None, :, :]
        ibc = jax.nn.relu(jax.lax.dot_general(masked, Wb_ref[...], _MM_TT,
                                              preferred_element_type=f32))
        ib_ref[pl.ds(r, C)] = ibc
        ibT_ref[:, pl.ds(r, C), :] = jnp.transpose(ibc, (1, 0, 2))
        return A0 + jnp.sum(ibc, axis=0), jnp.maximum(X0, jnp.max(ibc, axis=0))

    A0 = jnp.zeros((N, H), f32)
    X0 = jnp.full((N, H), -jnp.inf, f32)
    A0, X0 = jax.lax.fori_loop(0, NCH, p1, (A0, X0))

    @pl.when(g + 2 < ng)
    def _():
        fb_copy(g + 2, slot).start()

    ma = ia + A0 * jax.nn.sigmoid(X0)
    ma_ref[...] = ma

    Gm = jax.lax.dot_general(ma, ma, _MM_GRAM, preferred_element_type=f32)
    RT = Gm * adjT_ref[...]
    STm = jnp.max(RT, axis=1, keepdims=True)
    STe = jnp.exp(RT - STm)
    st_ref[...] = STe / jnp.sum(STe, axis=1, keepdims=True)

    def it1(ci, _):
        r = ci * C
        adjc = adj_ref[0, pl.ds(r, C)]
        ibc = ib_ref[pl.ds(r, C)]
        DT = adjc[:, :, None] * ma_ref[...][None, :, :] - ibc
        mm = jax.lax.dot_general(DT.reshape(C * N, H), W0_ref[...], _MM_NT,
                                 preferred_element_type=f32).reshape(C, N, H)
        out = jax.nn.relu(ibT_ref[pl.ds(r, C)] + mm) * st_ref[pl.ds(r, C)][:, :, None]
        mT_ref[pl.ds(r, C)] = out
        acc_ref[pl.ds(r, C)] = jnp.sum(out, axis=1)
        xmx_ref[pl.ds(r, C)] = jnp.max(out, axis=1)
        return 0

    jax.lax.fori_loop(0, NCH, it1, 0)

    ma2 = ma_ref[...] + acc_ref[...] * jax.nn.sigmoid(xmx_ref[...])
    ma_ref[...] = ma2

    G2 = jax.lax.dot_general(ma2, ma2, _MM_GRAM, preferred_element_type=f32)
    R2 = G2 * adj_ref[0]
    S2m = jnp.max(R2, axis=0, keepdims=True)
    S2e = jnp.exp(R2 - S2m)
    st_ref[...] = S2e / jnp.sum(S2e, axis=0, keepdims=True)

    def it2(ci, carry):
        A2, X2 = carry
        r = ci * C
        adjTc = adjT_ref[pl.ds(r, C)]
        ma2c = ma_ref[pl.ds(r, C)]
        D2 = adjTc[:, :, None] * ma2c[:, None, :] - mT_ref[pl.ds(r, C)]
        mm = jax.lax.dot_general(D2.reshape(C * N, H), W1_ref[...], _MM_NT,
                                 preferred_element_type=f32).reshape(C, N, H)
        out = jax.nn.relu(ib_ref[pl.ds(r, C)] + mm) * st_ref[pl.ds(r, C)][:, :, None]
        mb_ref[0, pl.ds(r, C)] = jnp.swapaxes(out, 1, 2)
        return A2 + jnp.sum(out, axis=0), jnp.maximum(X2, jnp.max(out, axis=0))

    A2 = jnp.zeros((N, H), f32)
    X2 = jnp.full((N, H), -jnp.inf, f32)
    A2, X2 = jax.lax.fori_loop(0, NCH, it2, (A2, X2))

    agg2 = A2 * jax.nn.sigmoid(X2)

    Wo = Wo_ref[...]
    pre_o = (jax.lax.dot_general(agg2, Wo[:, 0:H], _MM_NT, preferred_element_type=f32)
             + jax.lax.dot_general(ma_ref[...], Wo[:, H:2 * H], _MM_NT,
                                   preferred_element_type=f32)
             + jax.lax.dot_general(ia_ref[...], Wo[:, 2 * H:3 * H], _MM_NT,
                                   preferred_element_type=f32)
             + bo_ref[...])
    ah_ref[0] = jax.nn.relu(pre_o).T


@jax.jit
def kernel(f_atoms, f_bonds, adj, W_i_atom, W_i_bond, W_h_0, W_h_1, W_o, b_o):
    B, S, n, AF = f_atoms.shape
    BF = f_bonds.shape[-1]
    G = B * S
    fa = f_atoms.reshape(G, n, AF)
    fbT = jnp.transpose(f_bonds, (0, 1, 4, 2, 3)).reshape(G, BF, n, n)
    a = adj.reshape(G, n, n)
    ah, mb = pl.pallas_call(
        _mpn_body,
        grid=(G,),
        in_specs=[
            pl.BlockSpec((1, N, AF), lambda g: (g, 0, 0)),
            pl.BlockSpec(memory_space=pltpu.MemorySpace.HBM),
            pl.BlockSpec((1, N, N), lambda g: (g, 0, 0)),
            pl.BlockSpec((H, AF), lambda g: (0, 0)),
            pl.BlockSpec((H, BF), lambda g: (0, 0)),
            pl.BlockSpec((H, H), lambda g: (0, 0)),
            pl.BlockSpec((H, H), lambda g: (0, 0)),
            pl.BlockSpec((H, 3 * H), lambda g: (0, 0)),
            pl.BlockSpec((1, H), lambda g: (0, 0)),
        ],
        out_specs=[
            pl.BlockSpec((1, H, N), lambda g: (g, 0, 0)),
            pl.BlockSpec((1, N, H, N), lambda g: (g, 0, 0, 0)),
        ],
        out_shape=[
            jax.ShapeDtypeStruct((G, H, N), jnp.float32),
            jax.ShapeDtypeStruct((G, N, H, N), jnp.float32),
        ],
        scratch_shapes=[
            pltpu.VMEM((N, N, H), jnp.float32),
            pltpu.VMEM((N, N, H), jnp.float32),
            pltpu.VMEM((N, N, H), jnp.float32),
            pltpu.VMEM((N, N), jnp.float32),
            pltpu.VMEM((N, H), jnp.float32),
            pltpu.VMEM((N, H), jnp.float32),
            pltpu.VMEM((N, N), jnp.float32),
            pltpu.VMEM((N, H), jnp.float32),
            pltpu.VMEM((N, H), jnp.float32),
            pltpu.VMEM((2, BF, N, N), jnp.float32),
            pltpu.SemaphoreType.DMA((2,)),
        ],
    )(fa, fbT, a, W_i_atom, W_i_bond, W_h_0, W_h_1, W_o, b_o.reshape(1, H))
    ah_l = jnp.transpose(ah, (0, 2, 1)).reshape(B, S, n, H)
    mb_l = jnp.transpose(mb, (0, 1, 3, 2)).reshape(B, S, n, n, H)
    return ah_l, mb_l
